# Initial kernel scaffold; baseline (speedup 1.0000x reference)
import jax
import jax.numpy as jnp
from jax.experimental import pallas as pl


def kernel(H, edge_index, P_edge, deter_edge, W1, W2, W3, W4, Wv, Wout_w, Wout_b, res_w, res_b, ln_g, ln_b):
    raise NotImplementedError("write your pallas kernel here")


# speedup vs baseline: 4.4992x; 4.4992x over previous
import functools

import jax
import jax.numpy as jnp
from jax import lax
from jax.experimental import pallas as pl
from jax.experimental.pallas import tpu as pltpu
from jax.experimental.pallas import tpu_sc as plsc

NC = 2
NS = 16
LANES = 16
CHUNK = 80


def _t1_body(h_ref, w1_ref, w2_ref, w4x_ref, w3_ref, wv_ref,
             a1_ref, a2_ref, v_ref, w34_ref):
    h = h_ref[...]
    w14 = jnp.dot(w1_ref[...], w4x_ref[...], preferred_element_type=jnp.float32)
    w24 = jnp.dot(w2_ref[...], w4x_ref[...], preferred_element_type=jnp.float32)
    a1_ref[...] = jnp.dot(h, w14, preferred_element_type=jnp.float32)
    a2_ref[...] = jnp.dot(h, w24, preferred_element_type=jnp.float32)
    v_ref[...] = jnp.dot(h, wv_ref[...], preferred_element_type=jnp.float32)
    w34_ref[...] = jnp.dot(w3_ref[...], w4x_ref[...],
                           preferred_element_type=jnp.float32)


def _t1(H, W1, W2, W4x, W3, Wv):
    N, D = H.shape
    H2 = W4x.shape[1]
    BN = 1000
    grid = (N // BN,)
    f32 = jnp.float32
    return pl.pallas_call(
        _t1_body,
        grid=grid,
        in_specs=[
            pl.BlockSpec((BN, D), lambda i: (i, 0)),
            pl.BlockSpec((D, D), lambda i: (0, 0)),
            pl.BlockSpec((D, D), lambda i: (0, 0)),
            pl.BlockSpec((D, H2), lambda i: (0, 0)),
            pl.BlockSpec((1, D), lambda i: (0, 0)),
            pl.BlockSpec((D, D), lambda i: (0, 0)),
        ],
        out_specs=[
            pl.BlockSpec((BN, H2), lambda i: (i, 0)),
            pl.BlockSpec((BN, H2), lambda i: (i, 0)),
            pl.BlockSpec((BN, D), lambda i: (i, 0)),
            pl.BlockSpec((1, H2), lambda i: (0, 0)),
        ],
        out_shape=[
            jax.ShapeDtypeStruct((N, H2), f32),
            jax.ShapeDtypeStruct((N, H2), f32),
            jax.ShapeDtypeStruct((N, D), f32),
            jax.ShapeDtypeStruct((1, H2), f32),
        ],
    )(H, W1, W2, W4x, W3, Wv)


def _t2_body(agg_ref, h_ref, wout_ref, wres_ref, wb_ref, rb_ref, g_ref, b_ref,
             y_ref):
    a = agg_ref[0] + agg_ref[1]
    x = jnp.dot(a, wout_ref[...], preferred_element_type=jnp.float32)
    x = x + jnp.dot(h_ref[...], wres_ref[...], preferred_element_type=jnp.float32)
    x = x + wb_ref[...] + rb_ref[...]
    mu = jnp.mean(x, axis=-1, keepdims=True)
    xc = x - mu
    var = jnp.mean(xc * xc, axis=-1, keepdims=True)
    y_ref[...] = g_ref[...] * (xc * lax.rsqrt(var + 1e-5)) + b_ref[...]


def _t2(agg, H, Wout_w, res_w, wout_b, res_b, ln_g, ln_b):
    N, D = H.shape
    BN = 1000
    grid = (N // BN,)
    return pl.pallas_call(
        _t2_body,
        grid=grid,
        in_specs=[
            pl.BlockSpec((NC, BN, D), lambda i: (0, i, 0)),
            pl.BlockSpec((BN, D), lambda i: (i, 0)),
            pl.BlockSpec((D, D), lambda i: (0, 0)),
            pl.BlockSpec((D, D), lambda i: (0, 0)),
            pl.BlockSpec((1, D), lambda i: (0, 0)),
            pl.BlockSpec((1, D), lambda i: (0, 0)),
            pl.BlockSpec((1, D), lambda i: (0, 0)),
            pl.BlockSpec((1, D), lambda i: (0, 0)),
        ],
        out_specs=pl.BlockSpec((BN, D), lambda i: (i, 0)),
        out_shape=jax.ShapeDtypeStruct((N, D), jnp.float32),
    )(agg, H, Wout_w, res_w, wout_b, res_b, ln_g, ln_b)


def _sc1_body(Np, E, a1_hbm, a2_hbm, dst_hbm, src_hbm, p_hbm, det_hbm,
              w34_hbm, z16_hbm,
              exp_hbm, s0_hbm, s1_hbm,
              dstb, srcb, a1b, a2b, pb, detb, expb, w34v, sem, s_sh):
    cid = lax.axis_index("c")
    sid = lax.axis_index("s")
    e_per_w = E // (NC * NS)
    n_chunks = e_per_w // CHUNK
    base = (cid * NS + sid) * e_per_w
    rpt = Np // NS
    r0 = sid * rpt

    pltpu.sync_copy(z16_hbm.at[pl.ds(r0, rpt)], s_sh.at[pl.ds(r0, rpt)])
    pltpu.sync_copy(w34_hbm, w34v)
    plsc.subcore_barrier()

    def chunk_body(c, carry):
        cb = base + c * CHUNK
        pltpu.sync_copy(dst_hbm.at[pl.ds(cb, CHUNK)], dstb)
        pltpu.sync_copy(src_hbm.at[pl.ds(cb, CHUNK)], srcb)
        pltpu.sync_copy(p_hbm.at[pl.ds(cb, CHUNK)], pb)
        pltpu.sync_copy(det_hbm.at[pl.ds(cb, CHUNK)], detb)
        g1 = pltpu.async_copy(a1_hbm.at[dstb], a1b, sem)
        g2 = pltpu.async_copy(a2_hbm.at[srcb], a2b, sem)
        g1.wait()
        g2.wait()
        w34 = w34v[...]

        def edge_body(i, ecarry):
            iv = jnp.full((LANES,), i, jnp.int32)
            pv = plsc.load_gather(pb, [iv])
            dv = plsc.load_gather(detb, [iv])
            l = a1b[i, :] + a2b[i, :] + pv * w34 + dv
            l = jnp.where(l >= 0.0, l, 0.2 * l)
            expb[i, :] = jnp.exp(l)
            return ecarry

        lax.fori_loop(0, CHUNK, edge_body, 0)
        pltpu.sync_copy(expb, s_sh.at[dstb], add=True)
        pltpu.sync_copy(expb, exp_hbm.at[pl.ds(cb, CHUNK)])
        return carry

    lax.fori_loop(0, n_chunks, chunk_body, 0)
    plsc.subcore_barrier()

    @pl.when(cid == 0)
    def _():
        pltpu.sync_copy(s_sh.at[pl.ds(r0, rpt)], s0_hbm.at[pl.ds(r0, rpt)])

    @pl.when(cid == 1)
    def _():
        pltpu.sync_copy(s_sh.at[pl.ds(r0, rpt)], s1_hbm.at[pl.ds(r0, rpt)])


def _sc1(A1, A2, dst, src, P_edge, deter_edge, w34, z16):
    Np = z16.shape[0]
    E = dst.shape[0]
    H2 = A1.shape[1]
    f32 = jnp.float32
    mesh = plsc.VectorSubcoreMesh(core_axis_name="c", subcore_axis_name="s",
                                  num_cores=NC, num_subcores=NS)
    k = pl.kernel(
        functools.partial(_sc1_body, Np, E),
        out_type=(
            jax.ShapeDtypeStruct((E, H2), f32),
            jax.ShapeDtypeStruct((Np, H2), f32),
            jax.ShapeDtypeStruct((Np, H2), f32),
        ),
        mesh=mesh,
        compiler_params=pltpu.CompilerParams(needs_layout_passes=False, use_tc_tiling_on_sc=False),
        scratch_types=[
            pltpu.VMEM((CHUNK,), jnp.int32),
            pltpu.VMEM((CHUNK,), jnp.int32),
            pltpu.VMEM((CHUNK, H2), f32),
            pltpu.VMEM((CHUNK, H2), f32),
            pltpu.VMEM((CHUNK,), f32),
            pltpu.VMEM((CHUNK,), f32),
            pltpu.VMEM((CHUNK, H2), f32),
            pltpu.VMEM((LANES,), f32),
            pltpu.SemaphoreType.DMA,
            pltpu.VMEM_SHARED((Np, H2), f32),
        ],
    )
    return k(A1, A2, dst, src, P_edge, deter_edge, w34, z16)


def _sc2_body(Np, E, NH, HD, v_hbm, s0_hbm, s1_hbm, exp_hbm, dst_hbm, src_hbm,
              z128_hbm,
              agg_hbm,
              dstb, srcb, vrows, expb, s0b, s1b, sem, agg_sh):
    cid = lax.axis_index("c")
    sid = lax.axis_index("s")
    e_per_w = E // (NC * NS)
    n_chunks = e_per_w // CHUNK
    base = (cid * NS + sid) * e_per_w
    rpt = Np // NS
    r0 = sid * rpt

    pltpu.sync_copy(z128_hbm.at[pl.ds(r0, rpt)], agg_sh.at[pl.ds(r0, rpt)])
    plsc.subcore_barrier()

    def chunk_body(c, carry):
        cb = base + c * CHUNK
        pltpu.sync_copy(dst_hbm.at[pl.ds(cb, CHUNK)], dstb)
        pltpu.sync_copy(src_hbm.at[pl.ds(cb, CHUNK)], srcb)
        g1 = pltpu.async_copy(v_hbm.at[srcb], vrows, sem)
        g2 = pltpu.async_copy(s0_hbm.at[dstb], s0b, sem)
        g3 = pltpu.async_copy(s1_hbm.at[dstb], s1b, sem)
        pltpu.sync_copy(exp_hbm.at[pl.ds(cb, CHUNK)], expb)
        g1.wait()
        g2.wait()
        g3.wait()

        def edge_body(i, ecarry):
            alpha = expb[i, :] / (s0b[i, :] + s1b[i, :] + 1e-12)
            expb[i, :] = alpha
            iv = jnp.full((LANES,), i, jnp.int32)
            for h in range(NH):
                hv = jnp.full((LANES,), h, jnp.int32)
                av = plsc.load_gather(expb, [iv, hv])
                sl = vrows[i, pl.ds(h * HD, HD)]
                vrows[i, pl.ds(h * HD, HD)] = sl * av
            return ecarry

        lax.fori_loop(0, CHUNK, edge_body, 0)
        pltpu.sync_copy(vrows, agg_sh.at[dstb], add=True)
        return carry

    lax.fori_loop(0, n_chunks, chunk_body, 0)
    plsc.subcore_barrier()
    pltpu.sync_copy(agg_sh.at[pl.ds(r0, rpt)],
                    agg_hbm.at[cid, pl.ds(r0, rpt)])


def _sc2(V, s0, s1, exp_e, dst, src, z128):
    D = V.shape[1]
    Np = z128.shape[0]
    E = dst.shape[0]
    H2 = exp_e.shape[1]
    NH = H2 // 2
    HD = D // NH
    f32 = jnp.float32
    mesh = plsc.VectorSubcoreMesh(core_axis_name="c", subcore_axis_name="s",
                                  num_cores=NC, num_subcores=NS)
    k = pl.kernel(
        functools.partial(_sc2_body, Np, E, NH, HD),
        out_type=jax.ShapeDtypeStruct((NC, Np, D), f32),
        mesh=mesh,
        compiler_params=pltpu.CompilerParams(needs_layout_passes=False, use_tc_tiling_on_sc=False),
        scratch_types=[
            pltpu.VMEM((CHUNK,), jnp.int32),
            pltpu.VMEM((CHUNK,), jnp.int32),
            pltpu.VMEM((CHUNK, D), f32),
            pltpu.VMEM((CHUNK, H2), f32),
            pltpu.VMEM((CHUNK, H2), f32),
            pltpu.VMEM((CHUNK, H2), f32),
            pltpu.SemaphoreType.DMA,
            pltpu.VMEM_SHARED((Np, D), f32),
        ],
    )
    return k(V, s0, s1, exp_e, dst, src, z128)


def kernel(H, edge_index, P_edge, deter_edge, W1, W2, W3, W4, Wv,
           Wout_w, Wout_b, res_w, res_b, ln_g, ln_b):
    N, D = H.shape
    E = edge_index.shape[1]
    NH = W4.shape[1]
    H2 = 2 * NH
    f32 = jnp.float32

    src = edge_index[0]
    dst = edge_index[1]
    W4x = jnp.concatenate([W4, W4], axis=1)

    A1, A2, V, w34row = _t1(H, W1, W2, W4x, W3, Wv)
    w34 = w34row.reshape(H2)

    Np = -(-N // (NS * 8)) * (NS * 8)
    z16 = jnp.zeros((Np, H2), f32)
    z128 = jnp.zeros((Np, D), f32)

    exp_e, s0, s1 = _sc1(A1, A2, dst, src, P_edge, deter_edge, w34, z16)
    agg = _sc2(V, s0, s1, exp_e, dst, src, z128)

    y = _t2(agg, H, Wout_w, res_w, Wout_b.reshape(1, D), res_b.reshape(1, D),
            ln_g.reshape(1, D), ln_b.reshape(1, D))
    return y


# speedup vs baseline: 5.9821x; 1.3296x over previous
import functools

import jax
import jax.numpy as jnp
from jax import lax
from jax.experimental import pallas as pl
from jax.experimental.pallas import tpu as pltpu
from jax.experimental.pallas import tpu_sc as plsc

NC = 2
NS = 16
LANES = 16
CHUNK = 80


def _t1_body(h_ref, w1_ref, w2_ref, w4x_ref, w3_ref, wv_ref,
             a1_ref, a2_ref, v_ref, w34_ref):
    h = h_ref[...]
    w14 = jnp.dot(w1_ref[...], w4x_ref[...], preferred_element_type=jnp.float32)
    w24 = jnp.dot(w2_ref[...], w4x_ref[...], preferred_element_type=jnp.float32)
    a1_ref[...] = jnp.dot(h, w14, preferred_element_type=jnp.float32)
    a2_ref[...] = jnp.dot(h, w24, preferred_element_type=jnp.float32)
    v_ref[...] = jnp.dot(h, wv_ref[...], preferred_element_type=jnp.float32)
    w34_ref[...] = jnp.dot(w3_ref[...], w4x_ref[...],
                           preferred_element_type=jnp.float32)


def _t1(H, W1, W2, W4x, W3, Wv):
    N, D = H.shape
    H2 = W4x.shape[1]
    BN = 1000
    grid = (N // BN,)
    f32 = jnp.float32
    return pl.pallas_call(
        _t1_body,
        grid=grid,
        in_specs=[
            pl.BlockSpec((BN, D), lambda i: (i, 0)),
            pl.BlockSpec((D, D), lambda i: (0, 0)),
            pl.BlockSpec((D, D), lambda i: (0, 0)),
            pl.BlockSpec((D, H2), lambda i: (0, 0)),
            pl.BlockSpec((1, D), lambda i: (0, 0)),
            pl.BlockSpec((D, D), lambda i: (0, 0)),
        ],
        out_specs=[
            pl.BlockSpec((BN, H2), lambda i: (i, 0)),
            pl.BlockSpec((BN, H2), lambda i: (i, 0)),
            pl.BlockSpec((BN, D), lambda i: (i, 0)),
            pl.BlockSpec((1, H2), lambda i: (0, 0)),
        ],
        out_shape=[
            jax.ShapeDtypeStruct((N, H2), f32),
            jax.ShapeDtypeStruct((N, H2), f32),
            jax.ShapeDtypeStruct((N, D), f32),
            jax.ShapeDtypeStruct((1, H2), f32),
        ],
    )(H, W1, W2, W4x, W3, Wv)


def _t2_body(agg_ref, h_ref, wout_ref, wres_ref, wb_ref, rb_ref, g_ref, b_ref,
             y_ref):
    a = agg_ref[0] + agg_ref[1]
    x = jnp.dot(a, wout_ref[...], preferred_element_type=jnp.float32)
    x = x + jnp.dot(h_ref[...], wres_ref[...], preferred_element_type=jnp.float32)
    x = x + wb_ref[...] + rb_ref[...]
    mu = jnp.mean(x, axis=-1, keepdims=True)
    xc = x - mu
    var = jnp.mean(xc * xc, axis=-1, keepdims=True)
    y_ref[...] = g_ref[...] * (xc * lax.rsqrt(var + 1e-5)) + b_ref[...]


def _t2(agg, H, Wout_w, res_w, wout_b, res_b, ln_g, ln_b):
    N, D = H.shape
    BN = 1000
    grid = (N // BN,)
    return pl.pallas_call(
        _t2_body,
        grid=grid,
        in_specs=[
            pl.BlockSpec((NC, BN, D), lambda i: (0, i, 0)),
            pl.BlockSpec((BN, D), lambda i: (i, 0)),
            pl.BlockSpec((D, D), lambda i: (0, 0)),
            pl.BlockSpec((D, D), lambda i: (0, 0)),
            pl.BlockSpec((1, D), lambda i: (0, 0)),
            pl.BlockSpec((1, D), lambda i: (0, 0)),
            pl.BlockSpec((1, D), lambda i: (0, 0)),
            pl.BlockSpec((1, D), lambda i: (0, 0)),
        ],
        out_specs=pl.BlockSpec((BN, D), lambda i: (i, 0)),
        out_shape=jax.ShapeDtypeStruct((N, D), jnp.float32),
    )(agg, H, Wout_w, res_w, wout_b, res_b, ln_g, ln_b)


def _sc1_body(Np, E, a1_hbm, a2_hbm, dst_hbm, src_hbm, p_hbm, det_hbm,
              w34_hbm, z16_hbm,
              exp_hbm, s0_hbm, s1_hbm,
              dstb, srcb, a1b, a2b, pb, detb, expb, w34v, sem, s_sh):
    cid = lax.axis_index("c")
    sid = lax.axis_index("s")
    e_per_w = E // (NC * NS)
    n_chunks = e_per_w // CHUNK
    base = (cid * NS + sid) * e_per_w
    rpt = Np // NS
    r0 = sid * rpt

    pltpu.sync_copy(z16_hbm.at[pl.ds(r0, rpt)], s_sh.at[pl.ds(r0, rpt)])
    pltpu.sync_copy(w34_hbm, w34v)
    plsc.subcore_barrier()

    def chunk_body(c, carry):
        cb = base + c * CHUNK
        pltpu.sync_copy(dst_hbm.at[pl.ds(cb, CHUNK)], dstb)
        pltpu.sync_copy(src_hbm.at[pl.ds(cb, CHUNK)], srcb)
        pltpu.sync_copy(p_hbm.at[pl.ds(cb, CHUNK)], pb)
        pltpu.sync_copy(det_hbm.at[pl.ds(cb, CHUNK)], detb)
        g1 = pltpu.async_copy(a1_hbm.at[dstb], a1b, sem)
        g2 = pltpu.async_copy(a2_hbm.at[srcb], a2b, sem)
        g1.wait()
        g2.wait()
        w34 = w34v[...]

        def edge_body(i, ecarry):
            iv = jnp.full((LANES,), i, jnp.int32)
            pv = plsc.load_gather(pb, [iv])
            dv = plsc.load_gather(detb, [iv])
            l = a1b[i, :] + a2b[i, :] + pv * w34 + dv
            l = jnp.where(l >= 0.0, l, 0.2 * l)
            expb[i, :] = jnp.exp(l)
            return ecarry

        lax.fori_loop(0, CHUNK, edge_body, 0)
        pltpu.sync_copy(expb, s_sh.at[dstb], add=True)
        pltpu.sync_copy(expb, exp_hbm.at[pl.ds(cb, CHUNK)])
        return carry

    lax.fori_loop(0, n_chunks, chunk_body, 0)
    plsc.subcore_barrier()

    @pl.when(cid == 0)
    def _():
        pltpu.sync_copy(s_sh.at[pl.ds(r0, rpt)], s0_hbm.at[pl.ds(r0, rpt)])

    @pl.when(cid == 1)
    def _():
        pltpu.sync_copy(s_sh.at[pl.ds(r0, rpt)], s1_hbm.at[pl.ds(r0, rpt)])


def _sc1(A1, A2, dst, src, P_edge, deter_edge, w34, z16):
    Np = z16.shape[0]
    E = dst.shape[0]
    H2 = A1.shape[1]
    f32 = jnp.float32
    mesh = plsc.VectorSubcoreMesh(core_axis_name="c", subcore_axis_name="s",
                                  num_cores=NC, num_subcores=NS)
    k = pl.kernel(
        functools.partial(_sc1_body, Np, E),
        out_type=(
            jax.ShapeDtypeStruct((E, H2), f32),
            jax.ShapeDtypeStruct((Np, H2), f32),
            jax.ShapeDtypeStruct((Np, H2), f32),
        ),
        mesh=mesh,
        compiler_params=pltpu.CompilerParams(needs_layout_passes=False, use_tc_tiling_on_sc=False),
        scratch_types=[
            pltpu.VMEM((CHUNK,), jnp.int32),
            pltpu.VMEM((CHUNK,), jnp.int32),
            pltpu.VMEM((CHUNK, H2), f32),
            pltpu.VMEM((CHUNK, H2), f32),
            pltpu.VMEM((CHUNK,), f32),
            pltpu.VMEM((CHUNK,), f32),
            pltpu.VMEM((CHUNK, H2), f32),
            pltpu.VMEM((LANES,), f32),
            pltpu.SemaphoreType.DMA,
            pltpu.VMEM_SHARED((Np, H2), f32),
        ],
    )
    return k(A1, A2, dst, src, P_edge, deter_edge, w34, z16)


def _sc2_body(Np, E, NH, HD, v_hbm, s0_hbm, s1_hbm, exp_hbm, dst_hbm, src_hbm,
              z128_hbm,
              agg_hbm,
              dstb, srcb, vrows, expb, s0b, s1b, sem, agg_sh):
    cid = lax.axis_index("c")
    sid = lax.axis_index("s")
    e_per_w = E // (NC * NS)
    n_chunks = e_per_w // CHUNK
    base = (cid * NS + sid) * e_per_w
    rpt = Np // NS
    r0 = sid * rpt

    pltpu.sync_copy(z128_hbm.at[pl.ds(r0, rpt)], agg_sh.at[pl.ds(r0, rpt)])
    plsc.subcore_barrier()

    def chunk_body(c, carry):
        cb = base + c * CHUNK
        pltpu.sync_copy(dst_hbm.at[pl.ds(cb, CHUNK)], dstb)
        pltpu.sync_copy(src_hbm.at[pl.ds(cb, CHUNK)], srcb)
        g1 = pltpu.async_copy(v_hbm.at[srcb], vrows, sem)
        g2 = pltpu.async_copy(s0_hbm.at[dstb], s0b, sem)
        g3 = pltpu.async_copy(s1_hbm.at[dstb], s1b, sem)
        pltpu.sync_copy(exp_hbm.at[pl.ds(cb, CHUNK)], expb)
        g1.wait()
        g2.wait()
        g3.wait()

        def edge_body(i, ecarry):
            alpha = expb[i, :] / (s0b[i, :] + s1b[i, :] + 1e-12)
            for j in range(NH):
                sl = vrows[i, pl.ds(j * HD, HD)]
                vrows[i, pl.ds(j * HD, HD)] = sl * alpha
            return ecarry

        lax.fori_loop(0, CHUNK, edge_body, 0)
        pltpu.sync_copy(vrows, agg_sh.at[dstb], add=True)
        return carry

    lax.fori_loop(0, n_chunks, chunk_body, 0)
    plsc.subcore_barrier()
    pltpu.sync_copy(agg_sh.at[pl.ds(r0, rpt)],
                    agg_hbm.at[cid, pl.ds(r0, rpt)])


def _sc2(V, s0, s1, exp_e, dst, src, z128):
    D = V.shape[1]
    Np = z128.shape[0]
    E = dst.shape[0]
    H2 = exp_e.shape[1]
    NH = H2 // 2
    HD = D // NH
    f32 = jnp.float32
    mesh = plsc.VectorSubcoreMesh(core_axis_name="c", subcore_axis_name="s",
                                  num_cores=NC, num_subcores=NS)
    k = pl.kernel(
        functools.partial(_sc2_body, Np, E, NH, HD),
        out_type=jax.ShapeDtypeStruct((NC, Np, D), f32),
        mesh=mesh,
        compiler_params=pltpu.CompilerParams(needs_layout_passes=False, use_tc_tiling_on_sc=False),
        scratch_types=[
            pltpu.VMEM((CHUNK,), jnp.int32),
            pltpu.VMEM((CHUNK,), jnp.int32),
            pltpu.VMEM((CHUNK, D), f32),
            pltpu.VMEM((CHUNK, H2), f32),
            pltpu.VMEM((CHUNK, H2), f32),
            pltpu.VMEM((CHUNK, H2), f32),
            pltpu.SemaphoreType.DMA,
            pltpu.VMEM_SHARED((Np, D), f32),
        ],
    )
    return k(V, s0, s1, exp_e, dst, src, z128)


def kernel(H, edge_index, P_edge, deter_edge, W1, W2, W3, W4, Wv,
           Wout_w, Wout_b, res_w, res_b, ln_g, ln_b):
    N, D = H.shape
    E = edge_index.shape[1]
    NH = W4.shape[1]
    H2 = 2 * NH
    f32 = jnp.float32

    src = edge_index[0]
    dst = edge_index[1]
    W4x = jnp.concatenate([W4, W4], axis=1)
    HD = D // NH
    perm = (jnp.arange(D) % NH) * HD + jnp.arange(D) // NH
    Wv_p = Wv[:, perm]
    Wout_p = Wout_w[perm, :]

    A1, A2, V, w34row = _t1(H, W1, W2, W4x, W3, Wv_p)
    w34 = w34row.reshape(H2)

    Np = -(-N // (NS * 8)) * (NS * 8)
    z16 = jnp.zeros((Np, H2), f32)
    z128 = jnp.zeros((Np, D), f32)

    exp_e, s0, s1 = _sc1(A1, A2, dst, src, P_edge, deter_edge, w34, z16)
    agg = _sc2(V, s0, s1, exp_e, dst, src, z128)

    y = _t2(agg, H, Wout_p, res_w, Wout_b.reshape(1, D), res_b.reshape(1, D),
            ln_g.reshape(1, D), ln_b.reshape(1, D))
    return y


# speedup vs baseline: 8.0530x; 1.3462x over previous
import functools

import jax
import jax.numpy as jnp
from jax import lax
from jax.experimental import pallas as pl
from jax.experimental.pallas import tpu as pltpu
from jax.experimental.pallas import tpu_sc as plsc

NC = 2
NS = 16
LANES = 16
CHUNK = 200


def _t1_body(h_ref, w1_ref, w2_ref, w4x_ref, w3_ref, wv_ref,
             a1_ref, a2_ref, v_ref, w34_ref):
    h = h_ref[...]
    w14 = jnp.dot(w1_ref[...], w4x_ref[...], preferred_element_type=jnp.float32)
    w24 = jnp.dot(w2_ref[...], w4x_ref[...], preferred_element_type=jnp.float32)
    a1_ref[...] = jnp.dot(h, w14, preferred_element_type=jnp.float32)
    a2_ref[...] = jnp.dot(h, w24, preferred_element_type=jnp.float32)
    v_ref[...] = jnp.dot(h, wv_ref[...], preferred_element_type=jnp.float32)
    w34_ref[...] = jnp.dot(w3_ref[...], w4x_ref[...],
                           preferred_element_type=jnp.float32)


def _t1(H, W1, W2, W4x, W3, Wv):
    N, D = H.shape
    H2 = W4x.shape[1]
    BN = 1000
    grid = (N // BN,)
    f32 = jnp.float32
    return pl.pallas_call(
        _t1_body,
        grid=grid,
        in_specs=[
            pl.BlockSpec((BN, D), lambda i: (i, 0)),
            pl.BlockSpec((D, D), lambda i: (0, 0)),
            pl.BlockSpec((D, D), lambda i: (0, 0)),
            pl.BlockSpec((D, H2), lambda i: (0, 0)),
            pl.BlockSpec((1, D), lambda i: (0, 0)),
            pl.BlockSpec((D, D), lambda i: (0, 0)),
        ],
        out_specs=[
            pl.BlockSpec((BN, H2), lambda i: (i, 0)),
            pl.BlockSpec((BN, H2), lambda i: (i, 0)),
            pl.BlockSpec((BN, D), lambda i: (i, 0)),
            pl.BlockSpec((1, H2), lambda i: (0, 0)),
        ],
        out_shape=[
            jax.ShapeDtypeStruct((N, H2), f32),
            jax.ShapeDtypeStruct((N, H2), f32),
            jax.ShapeDtypeStruct((N, D), f32),
            jax.ShapeDtypeStruct((1, H2), f32),
        ],
    )(H, W1, W2, W4x, W3, Wv)


def _t2_body(agg_ref, h_ref, wout_ref, wres_ref, wb_ref, rb_ref, g_ref, b_ref,
             y_ref):
    a = agg_ref[0] + agg_ref[1]
    x = jnp.dot(a, wout_ref[...], preferred_element_type=jnp.float32)
    x = x + jnp.dot(h_ref[...], wres_ref[...], preferred_element_type=jnp.float32)
    x = x + wb_ref[...] + rb_ref[...]
    mu = jnp.mean(x, axis=-1, keepdims=True)
    xc = x - mu
    var = jnp.mean(xc * xc, axis=-1, keepdims=True)
    y_ref[...] = g_ref[...] * (xc * lax.rsqrt(var + 1e-5)) + b_ref[...]


def _t2(agg, H, Wout_w, res_w, wout_b, res_b, ln_g, ln_b):
    N, D = H.shape
    BN = 1000
    grid = (N // BN,)
    return pl.pallas_call(
        _t2_body,
        grid=grid,
        in_specs=[
            pl.BlockSpec((NC, BN, D), lambda i: (0, i, 0)),
            pl.BlockSpec((BN, D), lambda i: (i, 0)),
            pl.BlockSpec((D, D), lambda i: (0, 0)),
            pl.BlockSpec((D, D), lambda i: (0, 0)),
            pl.BlockSpec((1, D), lambda i: (0, 0)),
            pl.BlockSpec((1, D), lambda i: (0, 0)),
            pl.BlockSpec((1, D), lambda i: (0, 0)),
            pl.BlockSpec((1, D), lambda i: (0, 0)),
        ],
        out_specs=pl.BlockSpec((BN, D), lambda i: (i, 0)),
        out_shape=jax.ShapeDtypeStruct((N, D), jnp.float32),
    )(agg, H, Wout_w, res_w, wout_b, res_b, ln_g, ln_b)


def _sc1_body(Np, E, a1_hbm, a2_hbm, dst_hbm, src_hbm, p_hbm, det_hbm,
              w34_hbm, z16_hbm,
              exp_hbm, s0_hbm, s1_hbm,
              dstb, srcb, a1b, a2b, pb, detb, expb, w34v, sem, s_sh):
    cid = lax.axis_index("c")
    sid = lax.axis_index("s")
    e_per_w = E // (NC * NS)
    n_chunks = e_per_w // CHUNK
    base = (cid * NS + sid) * e_per_w
    rpt = Np // NS
    r0 = sid * rpt

    pltpu.sync_copy(z16_hbm.at[pl.ds(r0, rpt)], s_sh.at[pl.ds(r0, rpt)])
    pltpu.sync_copy(w34_hbm, w34v)
    plsc.subcore_barrier()

    def chunk_body(c, carry):
        cb = base + c * CHUNK
        pltpu.sync_copy(dst_hbm.at[pl.ds(cb, CHUNK)], dstb)
        pltpu.sync_copy(src_hbm.at[pl.ds(cb, CHUNK)], srcb)
        pltpu.sync_copy(p_hbm.at[pl.ds(cb, CHUNK)], pb)
        pltpu.sync_copy(det_hbm.at[pl.ds(cb, CHUNK)], detb)
        g1 = pltpu.async_copy(a1_hbm.at[dstb], a1b, sem)
        g2 = pltpu.async_copy(a2_hbm.at[srcb], a2b, sem)
        g1.wait()
        g2.wait()
        w34 = w34v[...]

        def edge_body(i, ecarry):
            iv = jnp.full((LANES,), i, jnp.int32)
            pv = plsc.load_gather(pb, [iv])
            dv = plsc.load_gather(detb, [iv])
            l = a1b[i, :] + a2b[i, :] + pv * w34 + dv
            l = jnp.where(l >= 0.0, l, 0.2 * l)
            expb[i, :] = jnp.exp(l)
            return ecarry

        lax.fori_loop(0, CHUNK, edge_body, 0)
        pltpu.sync_copy(expb, s_sh.at[dstb], add=True)
        pltpu.sync_copy(expb, exp_hbm.at[pl.ds(cb, CHUNK)])
        return carry

    lax.fori_loop(0, n_chunks, chunk_body, 0)
    plsc.subcore_barrier()

    @pl.when(cid == 0)
    def _():
        pltpu.sync_copy(s_sh.at[pl.ds(r0, rpt)], s0_hbm.at[pl.ds(r0, rpt)])

    @pl.when(cid == 1)
    def _():
        pltpu.sync_copy(s_sh.at[pl.ds(r0, rpt)], s1_hbm.at[pl.ds(r0, rpt)])


def _sc1(A1, A2, dst, src, P_edge, deter_edge, w34, z16):
    Np = z16.shape[0]
    E = dst.shape[0]
    H2 = A1.shape[1]
    f32 = jnp.float32
    mesh = plsc.VectorSubcoreMesh(core_axis_name="c", subcore_axis_name="s",
                                  num_cores=NC, num_subcores=NS)
    k = pl.kernel(
        functools.partial(_sc1_body, Np, E),
        out_type=(
            jax.ShapeDtypeStruct((E, H2), f32),
            jax.ShapeDtypeStruct((Np, H2), f32),
            jax.ShapeDtypeStruct((Np, H2), f32),
        ),
        mesh=mesh,
        compiler_params=pltpu.CompilerParams(needs_layout_passes=False, use_tc_tiling_on_sc=False),
        scratch_types=[
            pltpu.VMEM((CHUNK,), jnp.int32),
            pltpu.VMEM((CHUNK,), jnp.int32),
            pltpu.VMEM((CHUNK, H2), f32),
            pltpu.VMEM((CHUNK, H2), f32),
            pltpu.VMEM((CHUNK,), f32),
            pltpu.VMEM((CHUNK,), f32),
            pltpu.VMEM((CHUNK, H2), f32),
            pltpu.VMEM((LANES,), f32),
            pltpu.SemaphoreType.DMA,
            pltpu.VMEM_SHARED((Np, H2), f32),
        ],
    )
    return k(A1, A2, dst, src, P_edge, deter_edge, w34, z16)


def _sc2_body(Np, E, NH, HD, v_hbm, s0_hbm, s1_hbm, exp_hbm, dst_hbm, src_hbm,
              z128_hbm,
              agg_hbm,
              dstb, srcb, vrows, expb, s0b, s1b, sem, agg_sh):
    cid = lax.axis_index("c")
    sid = lax.axis_index("s")
    e_per_w = E // (NC * NS)
    n_chunks = e_per_w // CHUNK
    base = (cid * NS + sid) * e_per_w
    rpt = Np // NS
    r0 = sid * rpt

    pltpu.sync_copy(z128_hbm.at[pl.ds(r0, rpt)], agg_sh.at[pl.ds(r0, rpt)])
    plsc.subcore_barrier()

    def chunk_body(c, carry):
        cb = base + c * CHUNK
        pltpu.sync_copy(dst_hbm.at[pl.ds(cb, CHUNK)], dstb)
        pltpu.sync_copy(src_hbm.at[pl.ds(cb, CHUNK)], srcb)
        g1 = pltpu.async_copy(v_hbm.at[srcb], vrows, sem)
        g2 = pltpu.async_copy(s0_hbm.at[dstb], s0b, sem)
        g3 = pltpu.async_copy(s1_hbm.at[dstb], s1b, sem)
        pltpu.sync_copy(exp_hbm.at[pl.ds(cb, CHUNK)], expb)
        g1.wait()
        g2.wait()
        g3.wait()

        def edge_body(i, ecarry):
            alpha = expb[i, :] / (s0b[i, :] + s1b[i, :] + 1e-12)
            for j in range(NH):
                sl = vrows[i, pl.ds(j * HD, HD)]
                vrows[i, pl.ds(j * HD, HD)] = sl * alpha
            return ecarry

        lax.fori_loop(0, CHUNK, edge_body, 0)
        pltpu.sync_copy(vrows, agg_sh.at[dstb], add=True)
        return carry

    lax.fori_loop(0, n_chunks, chunk_body, 0)
    plsc.subcore_barrier()
    pltpu.sync_copy(agg_sh.at[pl.ds(r0, rpt)],
                    agg_hbm.at[cid, pl.ds(r0, rpt)])


def _sc2(V, s0, s1, exp_e, dst, src, z128):
    D = V.shape[1]
    Np = z128.shape[0]
    E = dst.shape[0]
    H2 = exp_e.shape[1]
    NH = H2 // 2
    HD = D // NH
    f32 = jnp.float32
    mesh = plsc.VectorSubcoreMesh(core_axis_name="c", subcore_axis_name="s",
                                  num_cores=NC, num_subcores=NS)
    k = pl.kernel(
        functools.partial(_sc2_body, Np, E, NH, HD),
        out_type=jax.ShapeDtypeStruct((NC, Np, D), f32),
        mesh=mesh,
        compiler_params=pltpu.CompilerParams(needs_layout_passes=False, use_tc_tiling_on_sc=False),
        scratch_types=[
            pltpu.VMEM((CHUNK,), jnp.int32),
            pltpu.VMEM((CHUNK,), jnp.int32),
            pltpu.VMEM((CHUNK, D), f32),
            pltpu.VMEM((CHUNK, H2), f32),
            pltpu.VMEM((CHUNK, H2), f32),
            pltpu.VMEM((CHUNK, H2), f32),
            pltpu.SemaphoreType.DMA,
            pltpu.VMEM_SHARED((Np, D), f32),
        ],
    )
    return k(V, s0, s1, exp_e, dst, src, z128)


def kernel(H, edge_index, P_edge, deter_edge, W1, W2, W3, W4, Wv,
           Wout_w, Wout_b, res_w, res_b, ln_g, ln_b):
    N, D = H.shape
    E = edge_index.shape[1]
    NH = W4.shape[1]
    H2 = 2 * NH
    f32 = jnp.float32

    src = edge_index[0]
    dst = edge_index[1]
    W4x = jnp.concatenate([W4, W4], axis=1)
    HD = D // NH
    perm = (jnp.arange(D) % NH) * HD + jnp.arange(D) // NH
    Wv_p = Wv[:, perm]
    Wout_p = Wout_w[perm, :]

    A1, A2, V, w34row = _t1(H, W1, W2, W4x, W3, Wv_p)
    w34 = w34row.reshape(H2)

    Np = -(-N // (NS * 8)) * (NS * 8)
    z16 = jnp.zeros((Np, H2), f32)
    z128 = jnp.zeros((Np, D), f32)

    exp_e, s0, s1 = _sc1(A1, A2, dst, src, P_edge, deter_edge, w34, z16)
    agg = _sc2(V, s0, s1, exp_e, dst, src, z128)

    y = _t2(agg, H, Wout_p, res_w, Wout_b.reshape(1, D), res_b.reshape(1, D),
            ln_g.reshape(1, D), ln_b.reshape(1, D))
    return y


# speedup vs baseline: 8.7015x; 1.0805x over previous
import functools

import jax
import jax.numpy as jnp
from jax import lax
from jax.experimental import pallas as pl
from jax.experimental.pallas import tpu as pltpu
from jax.experimental.pallas import tpu_sc as plsc

NC = 2
NS = 16
LANES = 16
CHUNK1 = 400
CHUNK2 = 200


def _t1_body(h_ref, w1_ref, w2_ref, w4x_ref, w3_ref, wv_ref,
             a1_ref, a2_ref, v_ref, w34_ref):
    h = h_ref[...]
    w14 = jnp.dot(w1_ref[...], w4x_ref[...], preferred_element_type=jnp.float32)
    w24 = jnp.dot(w2_ref[...], w4x_ref[...], preferred_element_type=jnp.float32)
    a1_ref[...] = jnp.dot(h, w14, preferred_element_type=jnp.float32)
    a2_ref[...] = jnp.dot(h, w24, preferred_element_type=jnp.float32)
    v_ref[...] = jnp.dot(h, wv_ref[...], preferred_element_type=jnp.float32)
    w34_ref[...] = jnp.dot(w3_ref[...], w4x_ref[...],
                           preferred_element_type=jnp.float32)


def _t1(H, W1, W2, W4x, W3, Wv):
    N, D = H.shape
    H2 = W4x.shape[1]
    BN = 1000
    grid = (N // BN,)
    f32 = jnp.float32
    return pl.pallas_call(
        _t1_body,
        grid=grid,
        in_specs=[
            pl.BlockSpec((BN, D), lambda i: (i, 0)),
            pl.BlockSpec((D, D), lambda i: (0, 0)),
            pl.BlockSpec((D, D), lambda i: (0, 0)),
            pl.BlockSpec((D, H2), lambda i: (0, 0)),
            pl.BlockSpec((1, D), lambda i: (0, 0)),
            pl.BlockSpec((D, D), lambda i: (0, 0)),
        ],
        out_specs=[
            pl.BlockSpec((BN, H2), lambda i: (i, 0)),
            pl.BlockSpec((BN, H2), lambda i: (i, 0)),
            pl.BlockSpec((BN, D), lambda i: (i, 0)),
            pl.BlockSpec((1, H2), lambda i: (0, 0)),
        ],
        out_shape=[
            jax.ShapeDtypeStruct((N, H2), f32),
            jax.ShapeDtypeStruct((N, H2), f32),
            jax.ShapeDtypeStruct((N, D), f32),
            jax.ShapeDtypeStruct((1, H2), f32),
        ],
    )(H, W1, W2, W4x, W3, Wv)


def _t2_body(agg_ref, h_ref, wout_ref, wres_ref, wb_ref, rb_ref, g_ref, b_ref,
             y_ref):
    a = agg_ref[0] + agg_ref[1]
    x = jnp.dot(a, wout_ref[...], preferred_element_type=jnp.float32)
    x = x + jnp.dot(h_ref[...], wres_ref[...], preferred_element_type=jnp.float32)
    x = x + wb_ref[...] + rb_ref[...]
    mu = jnp.mean(x, axis=-1, keepdims=True)
    xc = x - mu
    var = jnp.mean(xc * xc, axis=-1, keepdims=True)
    y_ref[...] = g_ref[...] * (xc * lax.rsqrt(var + 1e-5)) + b_ref[...]


def _t2(agg, H, Wout_w, res_w, wout_b, res_b, ln_g, ln_b):
    N, D = H.shape
    BN = 1000
    grid = (N // BN,)
    return pl.pallas_call(
        _t2_body,
        grid=grid,
        in_specs=[
            pl.BlockSpec((NC, BN, D), lambda i: (0, i, 0)),
            pl.BlockSpec((BN, D), lambda i: (i, 0)),
            pl.BlockSpec((D, D), lambda i: (0, 0)),
            pl.BlockSpec((D, D), lambda i: (0, 0)),
            pl.BlockSpec((1, D), lambda i: (0, 0)),
            pl.BlockSpec((1, D), lambda i: (0, 0)),
            pl.BlockSpec((1, D), lambda i: (0, 0)),
            pl.BlockSpec((1, D), lambda i: (0, 0)),
        ],
        out_specs=pl.BlockSpec((BN, D), lambda i: (i, 0)),
        out_shape=jax.ShapeDtypeStruct((N, D), jnp.float32),
    )(agg, H, Wout_w, res_w, wout_b, res_b, ln_g, ln_b)


def _sc1_body(Np, E, a1_hbm, a2_hbm, dst_hbm, src_hbm, p_hbm, det_hbm,
              w34_hbm, z16_hbm,
              exp_hbm, s0_hbm, s1_hbm,
              dstb, srcb, a1b, a2b, pb, detb, expb, w34v, sem, s_sh):
    cid = lax.axis_index("c")
    sid = lax.axis_index("s")
    e_per_w = E // (NC * NS)
    n_chunks = e_per_w // CHUNK1
    base = (cid * NS + sid) * e_per_w
    rpt = Np // NS
    r0 = sid * rpt

    pltpu.sync_copy(z16_hbm.at[pl.ds(r0, rpt)], s_sh.at[pl.ds(r0, rpt)])
    pltpu.sync_copy(w34_hbm, w34v)
    plsc.subcore_barrier()

    def chunk_body(c, carry):
        cb = base + c * CHUNK1
        pltpu.sync_copy(dst_hbm.at[pl.ds(cb, CHUNK1)], dstb)
        pltpu.sync_copy(src_hbm.at[pl.ds(cb, CHUNK1)], srcb)
        pltpu.sync_copy(p_hbm.at[pl.ds(cb, CHUNK1)], pb)
        pltpu.sync_copy(det_hbm.at[pl.ds(cb, CHUNK1)], detb)
        g1 = pltpu.async_copy(a1_hbm.at[dstb], a1b, sem)
        g2 = pltpu.async_copy(a2_hbm.at[srcb], a2b, sem)
        g1.wait()
        g2.wait()
        w34 = w34v[...]

        def edge_body(i, ecarry):
            iv = jnp.full((LANES,), i, jnp.int32)
            pv = plsc.load_gather(pb, [iv])
            dv = plsc.load_gather(detb, [iv])
            l = a1b[i, :] + a2b[i, :] + pv * w34 + dv
            l = jnp.where(l >= 0.0, l, 0.2 * l)
            expb[i, :] = jnp.exp(l)
            return ecarry

        lax.fori_loop(0, CHUNK1, edge_body, 0)
        pltpu.sync_copy(expb, s_sh.at[dstb], add=True)
        pltpu.sync_copy(expb, exp_hbm.at[pl.ds(cb, CHUNK1)])
        return carry

    lax.fori_loop(0, n_chunks, chunk_body, 0)
    plsc.subcore_barrier()

    @pl.when(cid == 0)
    def _():
        pltpu.sync_copy(s_sh.at[pl.ds(r0, rpt)], s0_hbm.at[pl.ds(r0, rpt)])

    @pl.when(cid == 1)
    def _():
        pltpu.sync_copy(s_sh.at[pl.ds(r0, rpt)], s1_hbm.at[pl.ds(r0, rpt)])


def _sc1(A1, A2, dst, src, P_edge, deter_edge, w34, z16):
    Np = z16.shape[0]
    E = dst.shape[0]
    H2 = A1.shape[1]
    f32 = jnp.float32
    mesh = plsc.VectorSubcoreMesh(core_axis_name="c", subcore_axis_name="s",
                                  num_cores=NC, num_subcores=NS)
    k = pl.kernel(
        functools.partial(_sc1_body, Np, E),
        out_type=(
            jax.ShapeDtypeStruct((E, H2), f32),
            jax.ShapeDtypeStruct((Np, H2), f32),
            jax.ShapeDtypeStruct((Np, H2), f32),
        ),
        mesh=mesh,
        compiler_params=pltpu.CompilerParams(needs_layout_passes=False, use_tc_tiling_on_sc=False),
        scratch_types=[
            pltpu.VMEM((CHUNK1,), jnp.int32),
            pltpu.VMEM((CHUNK1,), jnp.int32),
            pltpu.VMEM((CHUNK1, H2), f32),
            pltpu.VMEM((CHUNK1, H2), f32),
            pltpu.VMEM((CHUNK1,), f32),
            pltpu.VMEM((CHUNK1,), f32),
            pltpu.VMEM((CHUNK1, H2), f32),
            pltpu.VMEM((LANES,), f32),
            pltpu.SemaphoreType.DMA,
            pltpu.VMEM_SHARED((Np, H2), f32),
        ],
    )
    return k(A1, A2, dst, src, P_edge, deter_edge, w34, z16)


def _sc2_body(Np, E, NH, HD, v_hbm, s0_hbm, s1_hbm, exp_hbm, dst_hbm, src_hbm,
              z128_hbm,
              agg_hbm,
              dstb, srcb, vrows, expb, s0b, s1b, sem, agg_sh):
    cid = lax.axis_index("c")
    sid = lax.axis_index("s")
    e_per_w = E // (NC * NS)
    n_chunks = e_per_w // CHUNK2
    base = (cid * NS + sid) * e_per_w
    rpt = Np // NS
    r0 = sid * rpt

    pltpu.sync_copy(z128_hbm.at[pl.ds(r0, rpt)], agg_sh.at[pl.ds(r0, rpt)])
    plsc.subcore_barrier()

    def chunk_body(c, carry):
        cb = base + c * CHUNK2
        pltpu.sync_copy(dst_hbm.at[pl.ds(cb, CHUNK2)], dstb)
        pltpu.sync_copy(src_hbm.at[pl.ds(cb, CHUNK2)], srcb)
        g1 = pltpu.async_copy(v_hbm.at[srcb], vrows, sem)
        g2 = pltpu.async_copy(s0_hbm.at[dstb], s0b, sem)
        g3 = pltpu.async_copy(s1_hbm.at[dstb], s1b, sem)
        pltpu.sync_copy(exp_hbm.at[pl.ds(cb, CHUNK2)], expb)
        g1.wait()
        g2.wait()
        g3.wait()

        def edge_body(i, ecarry):
            alpha = expb[i, :] / (s0b[i, :] + s1b[i, :] + 1e-12)
            for j in range(NH):
                sl = vrows[i, pl.ds(j * HD, HD)]
                vrows[i, pl.ds(j * HD, HD)] = sl * alpha
            return ecarry

        lax.fori_loop(0, CHUNK2, edge_body, 0)
        pltpu.sync_copy(vrows, agg_sh.at[dstb], add=True)
        return carry

    lax.fori_loop(0, n_chunks, chunk_body, 0)
    plsc.subcore_barrier()
    pltpu.sync_copy(agg_sh.at[pl.ds(r0, rpt)],
                    agg_hbm.at[cid, pl.ds(r0, rpt)])


def _sc2(V, s0, s1, exp_e, dst, src, z128):
    D = V.shape[1]
    Np = z128.shape[0]
    E = dst.shape[0]
    H2 = exp_e.shape[1]
    NH = H2 // 2
    HD = D // NH
    f32 = jnp.float32
    mesh = plsc.VectorSubcoreMesh(core_axis_name="c", subcore_axis_name="s",
                                  num_cores=NC, num_subcores=NS)
    k = pl.kernel(
        functools.partial(_sc2_body, Np, E, NH, HD),
        out_type=jax.ShapeDtypeStruct((NC, Np, D), f32),
        mesh=mesh,
        compiler_params=pltpu.CompilerParams(needs_layout_passes=False, use_tc_tiling_on_sc=False),
        scratch_types=[
            pltpu.VMEM((CHUNK2,), jnp.int32),
            pltpu.VMEM((CHUNK2,), jnp.int32),
            pltpu.VMEM((CHUNK2, D), f32),
            pltpu.VMEM((CHUNK2, H2), f32),
            pltpu.VMEM((CHUNK2, H2), f32),
            pltpu.VMEM((CHUNK2, H2), f32),
            pltpu.SemaphoreType.DMA,
            pltpu.VMEM_SHARED((Np, D), f32),
        ],
    )
    return k(V, s0, s1, exp_e, dst, src, z128)


def kernel(H, edge_index, P_edge, deter_edge, W1, W2, W3, W4, Wv,
           Wout_w, Wout_b, res_w, res_b, ln_g, ln_b):
    N, D = H.shape
    E = edge_index.shape[1]
    NH = W4.shape[1]
    H2 = 2 * NH
    f32 = jnp.float32

    src = edge_index[0]
    dst = edge_index[1]
    W4x = jnp.concatenate([W4, W4], axis=1)
    HD = D // NH
    perm = (jnp.arange(D) % NH) * HD + jnp.arange(D) // NH
    Wv_p = Wv[:, perm]
    Wout_p = Wout_w[perm, :]

    A1, A2, V, w34row = _t1(H, W1, W2, W4x, W3, Wv_p)
    w34 = w34row.reshape(H2)

    Np = -(-N // (NS * 8)) * (NS * 8)
    z16 = jnp.zeros((Np, H2), f32)
    z128 = jnp.zeros((Np, D), f32)

    exp_e, s0, s1 = _sc1(A1, A2, dst, src, P_edge, deter_edge, w34, z16)
    agg = _sc2(V, s0, s1, exp_e, dst, src, z128)

    y = _t2(agg, H, Wout_p, res_w, Wout_b.reshape(1, D), res_b.reshape(1, D),
            ln_g.reshape(1, D), ln_b.reshape(1, D))
    return y


# speedup vs baseline: 9.3677x; 1.0766x over previous
import functools

import jax
import jax.numpy as jnp
from jax import lax
from jax.experimental import pallas as pl
from jax.experimental.pallas import tpu as pltpu
from jax.experimental.pallas import tpu_sc as plsc

NC = 2
NS = 16
LANES = 16
CHUNK1 = 2000
CHUNK2 = 200


def _t1_body(h_ref, w1_ref, w2_ref, w4x_ref, w3_ref, wv_ref,
             a1_ref, a2_ref, v_ref, w34_ref):
    h = h_ref[...]
    w14 = jnp.dot(w1_ref[...], w4x_ref[...], preferred_element_type=jnp.float32)
    w24 = jnp.dot(w2_ref[...], w4x_ref[...], preferred_element_type=jnp.float32)
    a1_ref[...] = jnp.dot(h, w14, preferred_element_type=jnp.float32)
    a2_ref[...] = jnp.dot(h, w24, preferred_element_type=jnp.float32)
    v_ref[...] = jnp.dot(h, wv_ref[...], preferred_element_type=jnp.float32)
    w34_ref[...] = jnp.dot(w3_ref[...], w4x_ref[...],
                           preferred_element_type=jnp.float32)


def _t1(H, W1, W2, W4x, W3, Wv):
    N, D = H.shape
    H2 = W4x.shape[1]
    BN = 1000
    grid = (N // BN,)
    f32 = jnp.float32
    return pl.pallas_call(
        _t1_body,
        grid=grid,
        in_specs=[
            pl.BlockSpec((BN, D), lambda i: (i, 0)),
            pl.BlockSpec((D, D), lambda i: (0, 0)),
            pl.BlockSpec((D, D), lambda i: (0, 0)),
            pl.BlockSpec((D, H2), lambda i: (0, 0)),
            pl.BlockSpec((1, D), lambda i: (0, 0)),
            pl.BlockSpec((D, D), lambda i: (0, 0)),
        ],
        out_specs=[
            pl.BlockSpec((BN, H2), lambda i: (i, 0)),
            pl.BlockSpec((BN, H2), lambda i: (i, 0)),
            pl.BlockSpec((BN, D), lambda i: (i, 0)),
            pl.BlockSpec((1, H2), lambda i: (0, 0)),
        ],
        out_shape=[
            jax.ShapeDtypeStruct((N, H2), f32),
            jax.ShapeDtypeStruct((N, H2), f32),
            jax.ShapeDtypeStruct((N, D), f32),
            jax.ShapeDtypeStruct((1, H2), f32),
        ],
    )(H, W1, W2, W4x, W3, Wv)


def _t2_body(agg_ref, h_ref, wout_ref, wres_ref, wb_ref, rb_ref, g_ref, b_ref,
             y_ref):
    a = agg_ref[0] + agg_ref[1]
    x = jnp.dot(a, wout_ref[...], preferred_element_type=jnp.float32)
    x = x + jnp.dot(h_ref[...], wres_ref[...], preferred_element_type=jnp.float32)
    x = x + wb_ref[...] + rb_ref[...]
    mu = jnp.mean(x, axis=-1, keepdims=True)
    xc = x - mu
    var = jnp.mean(xc * xc, axis=-1, keepdims=True)
    y_ref[...] = g_ref[...] * (xc * lax.rsqrt(var + 1e-5)) + b_ref[...]


def _t2(agg, H, Wout_w, res_w, wout_b, res_b, ln_g, ln_b):
    N, D = H.shape
    BN = 1000
    grid = (N // BN,)
    return pl.pallas_call(
        _t2_body,
        grid=grid,
        in_specs=[
            pl.BlockSpec((NC, BN, D), lambda i: (0, i, 0)),
            pl.BlockSpec((BN, D), lambda i: (i, 0)),
            pl.BlockSpec((D, D), lambda i: (0, 0)),
            pl.BlockSpec((D, D), lambda i: (0, 0)),
            pl.BlockSpec((1, D), lambda i: (0, 0)),
            pl.BlockSpec((1, D), lambda i: (0, 0)),
            pl.BlockSpec((1, D), lambda i: (0, 0)),
            pl.BlockSpec((1, D), lambda i: (0, 0)),
        ],
        out_specs=pl.BlockSpec((BN, D), lambda i: (i, 0)),
        out_shape=jax.ShapeDtypeStruct((N, D), jnp.float32),
    )(agg, H, Wout_w, res_w, wout_b, res_b, ln_g, ln_b)


def _sc1_body(Np, E, a1_hbm, a2_hbm, dst_hbm, src_hbm, p_hbm, det_hbm,
              w34_hbm, z16_hbm,
              exp_hbm, s0_hbm, s1_hbm,
              dstb, srcb, a1b, a2b, pb, detb, expb, w34v, sem, s_sh):
    cid = lax.axis_index("c")
    sid = lax.axis_index("s")
    e_per_w = E // (NC * NS)
    n_chunks = e_per_w // CHUNK1
    base = (cid * NS + sid) * e_per_w
    rpt = Np // NS
    r0 = sid * rpt

    pltpu.sync_copy(z16_hbm.at[pl.ds(r0, rpt)], s_sh.at[pl.ds(r0, rpt)])
    pltpu.sync_copy(w34_hbm, w34v)
    plsc.subcore_barrier()

    def chunk_body(c, carry):
        cb = base + c * CHUNK1
        pltpu.sync_copy(dst_hbm.at[pl.ds(cb, CHUNK1)], dstb)
        pltpu.sync_copy(src_hbm.at[pl.ds(cb, CHUNK1)], srcb)
        pltpu.sync_copy(p_hbm.at[pl.ds(cb, CHUNK1)], pb)
        pltpu.sync_copy(det_hbm.at[pl.ds(cb, CHUNK1)], detb)
        g1 = pltpu.async_copy(a1_hbm.at[dstb], a1b, sem)
        g2 = pltpu.async_copy(a2_hbm.at[srcb], a2b, sem)
        g1.wait()
        g2.wait()
        w34 = w34v[...]

        def edge_body(i, ecarry):
            iv = jnp.full((LANES,), i, jnp.int32)
            pv = plsc.load_gather(pb, [iv])
            dv = plsc.load_gather(detb, [iv])
            l = a1b[i, :] + a2b[i, :] + pv * w34 + dv
            l = jnp.where(l >= 0.0, l, 0.2 * l)
            expb[i, :] = jnp.exp(l)
            return ecarry

        lax.fori_loop(0, CHUNK1, edge_body, 0)
        pltpu.sync_copy(expb, s_sh.at[dstb], add=True)
        pltpu.sync_copy(expb, exp_hbm.at[pl.ds(cb, CHUNK1)])
        return carry

    lax.fori_loop(0, n_chunks, chunk_body, 0)
    plsc.subcore_barrier()

    @pl.when(cid == 0)
    def _():
        pltpu.sync_copy(s_sh.at[pl.ds(r0, rpt)], s0_hbm.at[pl.ds(r0, rpt)])

    @pl.when(cid == 1)
    def _():
        pltpu.sync_copy(s_sh.at[pl.ds(r0, rpt)], s1_hbm.at[pl.ds(r0, rpt)])


def _sc1(A1, A2, dst, src, P_edge, deter_edge, w34, z16):
    Np = z16.shape[0]
    E = dst.shape[0]
    H2 = A1.shape[1]
    f32 = jnp.float32
    mesh = plsc.VectorSubcoreMesh(core_axis_name="c", subcore_axis_name="s",
                                  num_cores=NC, num_subcores=NS)
    k = pl.kernel(
        functools.partial(_sc1_body, Np, E),
        out_type=(
            jax.ShapeDtypeStruct((E, H2), f32),
            jax.ShapeDtypeStruct((Np, H2), f32),
            jax.ShapeDtypeStruct((Np, H2), f32),
        ),
        mesh=mesh,
        compiler_params=pltpu.CompilerParams(needs_layout_passes=False, use_tc_tiling_on_sc=False),
        scratch_types=[
            pltpu.VMEM((CHUNK1,), jnp.int32),
            pltpu.VMEM((CHUNK1,), jnp.int32),
            pltpu.VMEM((CHUNK1, H2), f32),
            pltpu.VMEM((CHUNK1, H2), f32),
            pltpu.VMEM((CHUNK1,), f32),
            pltpu.VMEM((CHUNK1,), f32),
            pltpu.VMEM((CHUNK1, H2), f32),
            pltpu.VMEM((LANES,), f32),
            pltpu.SemaphoreType.DMA,
            pltpu.VMEM_SHARED((Np, H2), f32),
        ],
    )
    return k(A1, A2, dst, src, P_edge, deter_edge, w34, z16)


def _sc2_body(Np, E, NH, HD, v_hbm, s0_hbm, s1_hbm, exp_hbm, dst_hbm, src_hbm,
              z128_hbm,
              agg_hbm,
              dstb, srcb, vrows, expb, s0b, s1b, sem, agg_sh):
    cid = lax.axis_index("c")
    sid = lax.axis_index("s")
    e_per_w = E // (NC * NS)
    n_chunks = e_per_w // CHUNK2
    base = (cid * NS + sid) * e_per_w
    rpt = Np // NS
    r0 = sid * rpt

    pltpu.sync_copy(z128_hbm.at[pl.ds(r0, rpt)], agg_sh.at[pl.ds(r0, rpt)])
    plsc.subcore_barrier()

    def chunk_body(c, carry):
        cb = base + c * CHUNK2
        pltpu.sync_copy(dst_hbm.at[pl.ds(cb, CHUNK2)], dstb)
        pltpu.sync_copy(src_hbm.at[pl.ds(cb, CHUNK2)], srcb)
        g1 = pltpu.async_copy(v_hbm.at[srcb], vrows, sem)
        g2 = pltpu.async_copy(s0_hbm.at[dstb], s0b, sem)
        g3 = pltpu.async_copy(s1_hbm.at[dstb], s1b, sem)
        pltpu.sync_copy(exp_hbm.at[pl.ds(cb, CHUNK2)], expb)
        g1.wait()
        g2.wait()
        g3.wait()

        def edge_body(i, ecarry):
            alpha = expb[i, :] / (s0b[i, :] + s1b[i, :] + 1e-12)
            for j in range(NH):
                sl = vrows[i, pl.ds(j * HD, HD)]
                vrows[i, pl.ds(j * HD, HD)] = sl * alpha
            return ecarry

        lax.fori_loop(0, CHUNK2, edge_body, 0)
        pltpu.sync_copy(vrows, agg_sh.at[dstb], add=True)
        return carry

    lax.fori_loop(0, n_chunks, chunk_body, 0)
    plsc.subcore_barrier()
    pltpu.sync_copy(agg_sh.at[pl.ds(r0, rpt)],
                    agg_hbm.at[cid, pl.ds(r0, rpt)])


def _sc2(V, s0, s1, exp_e, dst, src, z128):
    D = V.shape[1]
    Np = z128.shape[0]
    E = dst.shape[0]
    H2 = exp_e.shape[1]
    NH = H2 // 2
    HD = D // NH
    f32 = jnp.float32
    mesh = plsc.VectorSubcoreMesh(core_axis_name="c", subcore_axis_name="s",
                                  num_cores=NC, num_subcores=NS)
    k = pl.kernel(
        functools.partial(_sc2_body, Np, E, NH, HD),
        out_type=jax.ShapeDtypeStruct((NC, Np, D), f32),
        mesh=mesh,
        compiler_params=pltpu.CompilerParams(needs_layout_passes=False, use_tc_tiling_on_sc=False),
        scratch_types=[
            pltpu.VMEM((CHUNK2,), jnp.int32),
            pltpu.VMEM((CHUNK2,), jnp.int32),
            pltpu.VMEM((CHUNK2, D), f32),
            pltpu.VMEM((CHUNK2, H2), f32),
            pltpu.VMEM((CHUNK2, H2), f32),
            pltpu.VMEM((CHUNK2, H2), f32),
            pltpu.SemaphoreType.DMA,
            pltpu.VMEM_SHARED((Np, D), f32),
        ],
    )
    return k(V, s0, s1, exp_e, dst, src, z128)


def kernel(H, edge_index, P_edge, deter_edge, W1, W2, W3, W4, Wv,
           Wout_w, Wout_b, res_w, res_b, ln_g, ln_b):
    N, D = H.shape
    E = edge_index.shape[1]
    NH = W4.shape[1]
    H2 = 2 * NH
    f32 = jnp.float32

    src = edge_index[0]
    dst = edge_index[1]
    W4x = jnp.concatenate([W4, W4], axis=1)
    HD = D // NH
    perm = (jnp.arange(D) % NH) * HD + jnp.arange(D) // NH
    Wv_p = Wv[:, perm]
    Wout_p = Wout_w[perm, :]

    A1, A2, V, w34row = _t1(H, W1, W2, W4x, W3, Wv_p)
    w34 = w34row.reshape(H2)

    Np = -(-N // (NS * 8)) * (NS * 8)
    z16 = jnp.zeros((Np, H2), f32)
    z128 = jnp.zeros((Np, D), f32)

    exp_e, s0, s1 = _sc1(A1, A2, dst, src, P_edge, deter_edge, w34, z16)
    agg = _sc2(V, s0, s1, exp_e, dst, src, z128)

    y = _t2(agg, H, Wout_p, res_w, Wout_b.reshape(1, D), res_b.reshape(1, D),
            ln_g.reshape(1, D), ln_b.reshape(1, D))
    return y
